# Initial kernel scaffold; baseline (speedup 1.0000x reference)
import functools

import jax
import jax.numpy as jnp
from jax import lax
from jax.experimental import pallas as pl
from jax.experimental.pallas import tpu as pltpu

N = 50000
E = 800000
G = 128
NPAD = 50176
RB = 512
NBLK = NPAD // RB
F1 = 80
F2 = 320
F3 = 784


def _scales_body(degA_ref, degB_ref, binv_ref, dinv_ref, sdinv_ref):
    da = degA_ref[...]
    db = degB_ref[...]
    binv_ref[...] = jnp.where(da > 0, 1.0 / jnp.where(da > 0, da, 1.0), 0.0)
    dinv_ref[...] = jnp.where(db > 0, 1.0 / jnp.where(db > 0, db, 1.0), 0.0)
    sdinv_ref[...] = lax.rsqrt(da + 1.0)


def _scales(degA, degB):
    f = pl.pallas_call(
        _scales_body,
        out_shape=[jax.ShapeDtypeStruct((NPAD // 128, 128), jnp.float32)] * 3,
    )
    binv, dinv, sdinv = f(degA.reshape(NPAD // 128, 128),
                          degB.reshape(NPAD // 128, 128))
    return binv.reshape(NPAD), dinv.reshape(NPAD), sdinv.reshape(NPAD)


def _mm1_body(a_ref, W_ref, b_ref, o_ref):
    o_ref[...] = jnp.maximum(
        jnp.dot(a_ref[...], W_ref[...], preferred_element_type=jnp.float32)
        + b_ref[0:1, :], 0.0)


def _mm1(a1, Wp, bp):
    return pl.pallas_call(
        _mm1_body,
        grid=(NBLK,),
        in_specs=[
            pl.BlockSpec((RB, F1), lambda i: (i, 0)),
            pl.BlockSpec((F1, F1), lambda i: (0, 0)),
            pl.BlockSpec((8, F1), lambda i: (0, 0)),
        ],
        out_specs=pl.BlockSpec((RB, F1), lambda i: (i, 0)),
        out_shape=jax.ShapeDtypeStruct((NPAD, F1), jnp.float32),
    )(a1, Wp, bp)


def _mm2_body(a_ref, s_ref, W_ref, b_ref, o0, o1, o2, o3):
    h = jnp.maximum(
        jnp.dot(a_ref[...], W_ref[...], preferred_element_type=jnp.float32)
        + b_ref[0:1, :], 0.0)
    h = h * s_ref[:, 0:1]
    o0[...] = h[:, 0:F1]
    o1[...] = h[:, F1:2 * F1]
    o2[...] = h[:, 2 * F1:3 * F1]
    o3[...] = h[:, 3 * F1:4 * F1]


def _mm2(a2, sdinv8, Wp, bp):
    return pl.pallas_call(
        _mm2_body,
        grid=(NBLK,),
        in_specs=[
            pl.BlockSpec((RB, F1), lambda i: (i, 0)),
            pl.BlockSpec((RB, 8), lambda i: (i, 0)),
            pl.BlockSpec((F1, F2), lambda i: (0, 0)),
            pl.BlockSpec((8, F2), lambda i: (0, 0)),
        ],
        out_specs=[pl.BlockSpec((RB, F1), lambda i: (i, 0))] * 4,
        out_shape=[jax.ShapeDtypeStruct((NPAD, F1), jnp.float32)] * 4,
    )(a2, sdinv8, Wp, bp)


def _mm3_body(a0, a1, a2, a3r, bt_ref, W_ref, b_ref, h3_ref, gs_ref, c_ref):
    i = pl.program_id(0)
    a = jnp.concatenate([a0[...], a1[...], a2[...], a3r[...]], axis=1)
    h = jnp.maximum(
        jnp.dot(a, W_ref[...], preferred_element_type=jnp.float32)
        + b_ref[0:1, :], 0.0)
    h3_ref[...] = h
    bt = bt_ref[:, 0:1]
    gids = lax.broadcasted_iota(jnp.int32, (RB, G), 1)
    mask = (bt == gids).astype(jnp.float32)

    @pl.when(i == 0)
    def _():
        gs_ref[...] = jnp.zeros_like(gs_ref)
        c_ref[...] = jnp.zeros_like(c_ref)

    gs_ref[...] += lax.dot_general(mask, h, (((0,), (0,)), ((), ())),
                                   preferred_element_type=jnp.float32)
    c_ref[...] += lax.dot_general(mask, jnp.ones((RB, 8), jnp.float32),
                                  (((0,), (0,)), ((), ())),
                                  preferred_element_type=jnp.float32)


def _mm3(a3s, batch8, Wp, bp):
    return pl.pallas_call(
        _mm3_body,
        grid=(NBLK,),
        in_specs=[pl.BlockSpec((RB, F1), lambda i: (i, 0))] * 4 + [
            pl.BlockSpec((RB, 8), lambda i: (i, 0)),
            pl.BlockSpec((F2, F3), lambda i: (0, 0)),
            pl.BlockSpec((8, F3), lambda i: (0, 0)),
        ],
        out_specs=[
            pl.BlockSpec((RB, F3), lambda i: (i, 0)),
            pl.BlockSpec((G, F3), lambda i: (0, 0)),
            pl.BlockSpec((G, 8), lambda i: (0, 0)),
        ],
        out_shape=[
            jax.ShapeDtypeStruct((NPAD, F3), jnp.float32),
            jax.ShapeDtypeStruct((G, F3), jnp.float32),
            jax.ShapeDtypeStruct((G, 8), jnp.float32),
        ],
    )(*a3s, batch8, Wp, bp)


def _head_body(gmx_ref, gs_ref, c_ref, f_ref, sb_ref, sa_ref, v_ref,
               Wg1, bg1, Wg2, bg2, Wf, bf, Wsb, bsb, Wsa, bsa, Wv, bv,
               W1, b1, W2, b2, Wo, bo, o_ref):
    gmax = jnp.maximum(gmx_ref[0], gmx_ref[1])
    cnt = c_ref[:, 0:1]
    gmean = gs_ref[...] / jnp.maximum(cnt, 1.0)
    g = jnp.concatenate([gmax[:, :780], gmean[:, :780]], axis=1)
    g = jnp.maximum(jnp.dot(g, Wg1[...], preferred_element_type=jnp.float32)
                    + bg1[0:1, :], 0.0)
    g = jnp.dot(g, Wg2[...], preferred_element_type=jnp.float32) + bg2[0:1, :]
    ff = jnp.dot(f_ref[...], Wf[...], preferred_element_type=jnp.float32) + bf[0:1, :]
    sb = jnp.dot(sb_ref[...], Wsb[...], preferred_element_type=jnp.float32) + bsb[0:1, :]
    sa = jnp.dot(sa_ref[...], Wsa[...], preferred_element_type=jnp.float32) + bsa[0:1, :]
    vv = jnp.dot(v_ref[...], Wv[...], preferred_element_type=jnp.float32) + bv[0:1, :]
    xc = jnp.concatenate([g, ff, sb, sa, vv], axis=1)
    xc = jnp.maximum(jnp.dot(xc, W1[...], preferred_element_type=jnp.float32)
                     + b1[0:1, :], 0.0)
    xc = jnp.maximum(jnp.dot(xc, W2[...], preferred_element_type=jnp.float32)
                     + b2[0:1, :], 0.0)
    o_ref[...] = jnp.dot(xc, Wo[...], preferred_element_type=jnp.float32) + bo[0:1, :]


def _head(gmax_parts, gsum, cnt8, fingerprint, seqbefore, seqafter, variant, wb):
    return pl.pallas_call(
        _head_body,
        out_shape=jax.ShapeDtypeStruct((G, 8), jnp.float32),
    )(gmax_parts, gsum, cnt8, fingerprint, seqbefore, seqafter, variant, *wb)


def _pad2(a, rows, cols):
    return jnp.pad(a, ((0, rows - a.shape[0]), (0, cols - a.shape[1])))


def _pad_bias(b, cols):
    return jnp.broadcast_to(jnp.pad(b, (0, cols - b.shape[0]))[None, :], (8, cols))


def _seg_sum(vals, idx, n):
    return jax.ops.segment_sum(vals, idx, num_segments=n)


def kernel(x, edge_index, batch, fingerprint, seqbefore, seqafter, variant,
           W_c1, b_c1, W_c2, b_c2, W_c3, b_c3, W_g1, b_g1, W_g2, b_g2,
           W_f, b_f, W_sb, b_sb, W_sa, b_sa, W_v, b_v,
           W_1, b_1, W_2, b_2, W_o, b_o):
    xp = _pad2(x, NPAD, F1)
    batchp = jnp.pad(batch, (0, NPAD - N), constant_values=G)
    batch8 = jnp.broadcast_to(batchp[:, None], (NPAD, 8))
    node_idx = edge_index[0]
    he_idx = edge_index[1]

    ones_e = jnp.ones(E, jnp.float32)
    degA = _seg_sum(ones_e, he_idx, NPAD)
    degB = _seg_sum(ones_e, node_idx, NPAD)
    binv, dinv, sdinv = _scales(degA, degB)
    sdinv8 = jnp.broadcast_to(sdinv[:, None], (NPAD, 8))

    def hyper_agg(v):
        e = _seg_sum(jnp.take(v, node_idx, axis=0), he_idx, NPAD) * binv[:, None]
        return _seg_sum(jnp.take(e, he_idx, axis=0), node_idx, NPAD) * dinv[:, None]

    a1 = hyper_agg(xp)
    h1 = _mm1(a1, _pad2(W_c1, F1, F1), _pad_bias(b_c1, F1))
    a2 = hyper_agg(h1)
    h2s = _mm2(a2, sdinv8, _pad2(W_c2, F1, F2), _pad_bias(b_c2, F2))
    a3s = []
    for p in range(4):
        agg = _seg_sum(jnp.take(h2s[p], node_idx, axis=0), he_idx, NPAD)
        a3s.append((agg + h2s[p]) * sdinv[:, None])
    h3, gsum, cnt8 = _mm3(a3s, batch8, _pad2(W_c3, F2, F3), _pad_bias(b_c3, F3))

    gm0 = jnp.maximum(
        jax.ops.segment_max(h3[:NPAD // 2], batchp[:NPAD // 2],
                            num_segments=G + 1)[:G], 0.0)
    gm1 = jnp.maximum(
        jax.ops.segment_max(h3[NPAD // 2:], batchp[NPAD // 2:],
                            num_segments=G + 1)[:G], 0.0)
    gmax_parts = jnp.stack([gm0, gm1])

    wb = [
        W_g1, _pad_bias(b_g1, 1500), W_g2, _pad_bias(b_g2, 128),
        W_f, _pad_bias(b_f, 128), W_sb, _pad_bias(b_sb, 128),
        W_sa, _pad_bias(b_sa, 128), W_v, _pad_bias(b_v, 384),
        W_1, _pad_bias(b_1, 512), W_2, _pad_bias(b_2, 128),
        _pad2(W_o, 128, 8), _pad_bias(b_o, 8),
    ]
    out = _head(gmax_parts, gsum, cnt8, fingerprint, seqbefore, seqafter,
                variant, wb)
    return out[:, :2]


# speedup vs baseline: 3.0288x; 3.0288x over previous
import jax
import jax.numpy as jnp
from jax import lax
from jax.experimental import pallas as pl
from jax.experimental.pallas import tpu as pltpu
from jax.experimental.pallas import tpu_sc as plsc

N = 50000
E = 800000
G = 128
NPAD = 50176
RB = 512
NBLK = NPAD // RB
F1 = 80
F1W = 80
F2 = 320
F3 = 784

NSC = 2
NTILE = 16
HALF = NPAD // 2
TROW = HALF // NTILE
EPAD = 801024
EPT = EPAD // NTILE
K = 48

_SC_MESH = dict(core_axis_name="c", subcore_axis_name="s")


def _agg_body(feat_hbm, scat_hbm, gath_hbm, out_hbm,
              sraw, graw, sidx, gidx, rows, zrow, acc, sem):
    c = lax.axis_index("c")
    s = lax.axis_index("s")
    ebase = s * EPT
    nbase = c * HALF
    lane = lax.iota(jnp.int32, 16)

    for r in range(16):
        for k0 in range(F1 // 16):
            zrow[r, pl.ds(k0 * 16, 16)] = jnp.zeros((16,), jnp.float32)

    def _z(i, _):
        pltpu.sync_copy(zrow, acc.at[pl.ds(s * TROW + i * 16, 16)])
        return 0
    lax.fori_loop(0, TROW // 16, _z, 0)
    plsc.subcore_barrier()

    def _chunk(j, _):
        pltpu.sync_copy(scat_hbm.at[pl.ds(ebase + j * K, K)], sraw)
        pltpu.sync_copy(gath_hbm.at[pl.ds(ebase + j * K, K)], graw)
        for u in range(K // 16):
            sk = sraw[pl.ds(u * 16, 16)]
            gk = graw[pl.ds(u * 16, 16)]
            loc = sk - nbase
            mi = lax.shift_right_arithmetic(
                jnp.bitwise_or(loc, (HALF - 1) - loc), 31) + 1
            sidx[pl.ds(u * 16, 16)] = jnp.minimum(
                jnp.maximum(loc, 0), HALF - 1)
            zsp = N + (j * 3 + u) % 11 * 16
            gidx[pl.ds(u * 16, 16)] = mi * gk + (1 - mi) * (zsp + lane)
        pltpu.async_copy(feat_hbm.at[gidx], rows, sem).wait()
        pltpu.sync_copy(rows, acc.at[sidx], add=True)
        return 0
    lax.fori_loop(0, EPT // K, _chunk, 0)
    plsc.subcore_barrier()

    g0 = c * HALF + s * TROW

    def _d(t, _):
        pltpu.sync_copy(acc.at[pl.ds(s * TROW + t * 16, 16)], zrow)
        pltpu.sync_copy(zrow, out_hbm.at[pl.ds(g0 + t * 16, 16)])
        return 0
    lax.fori_loop(0, TROW // 16, _d, 0)


def _agg(feat, scat, gath):
    f = pl.kernel(
        _agg_body,
        out_type=jax.ShapeDtypeStruct((NPAD, F1), jnp.float32),
        mesh=plsc.VectorSubcoreMesh(**_SC_MESH),
        compiler_params=pltpu.CompilerParams(use_tc_tiling_on_sc=False),
        scratch_types=[
            pltpu.VMEM((K,), jnp.int32),
            pltpu.VMEM((K,), jnp.int32),
            pltpu.VMEM((K,), jnp.int32),
            pltpu.VMEM((K,), jnp.int32),
            pltpu.VMEM((K, F1W), jnp.float32),
            pltpu.VMEM((16, F1), jnp.float32),
            pltpu.VMEM_SHARED((HALF, F1), jnp.float32),
            pltpu.SemaphoreType.DMA,
        ],
    )
    return f(feat, scat, gath)


def _pool_body(h3_hbm, batch_hbm, out_hbm, rowbuf, bbuf, acc):
    c = lax.axis_index("c")
    s = lax.axis_index("s")
    g0 = c * HALF + s * TROW

    def _z(i, _):
        acc[pl.ds(i * 16, 16)] = jnp.zeros((16,), jnp.float32)
        return 0
    lax.fori_loop(0, (G + 1) * F3 // 16, _z, 0)

    pltpu.sync_copy(batch_hbm.at[pl.ds(g0, TROW)], bbuf)

    def _grp(i, _):
        pltpu.sync_copy(h3_hbm.at[pl.ds(g0 + i * 16, 16)], rowbuf)
        bv = bbuf[pl.ds(i * 16, 16)]
        for r in range(16):
            base = bv[r] * F3

            def _col(k0, _):
                o = k0 * 16
                acc[pl.ds(base + o, 16)] = jnp.maximum(
                    acc[pl.ds(base + o, 16)], rowbuf[r, pl.ds(o, 16)])
                return 0
            lax.fori_loop(0, F3 // 16, _col, 0)
        return 0
    lax.fori_loop(0, TROW // 16, _grp, 0)

    pltpu.sync_copy(acc.at[pl.ds(0, G * F3)],
                    out_hbm.at[pl.ds((c * NTILE + s) * G * F3, G * F3)])


def _pool(h3, batchp):
    f = pl.kernel(
        _pool_body,
        out_type=jax.ShapeDtypeStruct((NSC * NTILE * G * F3,), jnp.float32),
        mesh=plsc.VectorSubcoreMesh(**_SC_MESH),
        compiler_params=pltpu.CompilerParams(use_tc_tiling_on_sc=False),
        scratch_types=[
            pltpu.VMEM((16, F3), jnp.float32),
            pltpu.VMEM((TROW,), jnp.int32),
            pltpu.VMEM(((G + 1) * F3,), jnp.float32),
        ],
    )
    return f(h3, batchp)


def _scales_body(degA_ref, degB_ref, binv_ref, dinv_ref, sdinv_ref):
    da = degA_ref[...]
    db = degB_ref[...]
    binv_ref[...] = jnp.where(da > 0, 1.0 / jnp.where(da > 0, da, 1.0), 0.0)
    dinv_ref[...] = jnp.where(db > 0, 1.0 / jnp.where(db > 0, db, 1.0), 0.0)
    sdinv_ref[...] = lax.rsqrt(da + 1.0)


def _scales(degA, degB):
    f = pl.pallas_call(
        _scales_body,
        out_shape=[jax.ShapeDtypeStruct((NPAD // 128, 128), jnp.float32)] * 3,
    )
    binv, dinv, sdinv = f(degA.reshape(NPAD // 128, 128),
                          degB.reshape(NPAD // 128, 128))
    return binv.reshape(NPAD), dinv.reshape(NPAD), sdinv.reshape(NPAD)


def _row_mask(i):
    rid = i * RB + lax.broadcasted_iota(jnp.int32, (RB, 1), 0)
    return (rid < N).astype(jnp.float32)


def _escale_body(a_ref, s_ref, o_ref):
    o_ref[...] = a_ref[...] * s_ref[:, 0:1]


def _escale(a, scale8):
    return pl.pallas_call(
        _escale_body,
        grid=(NBLK,),
        in_specs=[
            pl.BlockSpec((RB, F1), lambda i: (i, 0)),
            pl.BlockSpec((RB, 8), lambda i: (i, 0)),
        ],
        out_specs=pl.BlockSpec((RB, F1W), lambda i: (i, 0)),
        out_shape=jax.ShapeDtypeStruct((NPAD, F1W), jnp.float32),
    )(a, scale8)


def _mm1_body(a_ref, d_ref, W_ref, b_ref, o_ref):
    a = a_ref[...] * d_ref[:, 0:1]
    h = jnp.maximum(
        jnp.dot(a, W_ref[...], preferred_element_type=jnp.float32)
        + b_ref[0:1, :], 0.0)
    o_ref[...] = h * _row_mask(pl.program_id(0))


def _mm1(s2, dinv8, Wp, bp):
    return pl.pallas_call(
        _mm1_body,
        grid=(NBLK,),
        in_specs=[
            pl.BlockSpec((RB, F1), lambda i: (i, 0)),
            pl.BlockSpec((RB, 8), lambda i: (i, 0)),
            pl.BlockSpec((F1, F1W), lambda i: (0, 0)),
            pl.BlockSpec((8, F1W), lambda i: (0, 0)),
        ],
        out_specs=pl.BlockSpec((RB, F1W), lambda i: (i, 0)),
        out_shape=jax.ShapeDtypeStruct((NPAD, F1W), jnp.float32),
    )(s2, dinv8, Wp, bp)


def _mm2_body(a_ref, d_ref, s_ref, W_ref, b_ref, o0, o1, o2, o3):
    a = a_ref[...] * d_ref[:, 0:1]
    h = jnp.maximum(
        jnp.dot(a, W_ref[...], preferred_element_type=jnp.float32)
        + b_ref[0:1, :], 0.0)
    h = h * s_ref[:, 0:1] * _row_mask(pl.program_id(0))
    o0[...] = h[:, 0:F1]
    o1[...] = h[:, F1:2 * F1]
    o2[...] = h[:, 2 * F1:3 * F1]
    o3[...] = h[:, 3 * F1:4 * F1]


def _mm2(s4, dinv8, sdinv8, Wp, bp):
    return pl.pallas_call(
        _mm2_body,
        grid=(NBLK,),
        in_specs=[
            pl.BlockSpec((RB, F1), lambda i: (i, 0)),
            pl.BlockSpec((RB, 8), lambda i: (i, 0)),
            pl.BlockSpec((RB, 8), lambda i: (i, 0)),
            pl.BlockSpec((F1, F2), lambda i: (0, 0)),
            pl.BlockSpec((8, F2), lambda i: (0, 0)),
        ],
        out_specs=[pl.BlockSpec((RB, F1W), lambda i: (i, 0))] * 4,
        out_shape=[jax.ShapeDtypeStruct((NPAD, F1W), jnp.float32)] * 4,
    )(s4, dinv8, sdinv8, Wp, bp)


def _mm3_body(a0, a1, a2, a3r, h0, h1, h2, h3r, sd_ref, bt_ref, W_ref, b_ref,
              h3_ref, gs_ref, c_ref):
    i = pl.program_id(0)
    sd = sd_ref[:, 0:1]
    a = jnp.concatenate([(a0[...] + h0[...][:, :F1]) * sd,
                         (a1[...] + h1[...][:, :F1]) * sd,
                         (a2[...] + h2[...][:, :F1]) * sd,
                         (a3r[...] + h3r[...][:, :F1]) * sd], axis=1)
    h = jnp.maximum(
        jnp.dot(a, W_ref[...], preferred_element_type=jnp.float32)
        + b_ref[0:1, :], 0.0)
    h3_ref[...] = h
    bt = bt_ref[:, 0:1]
    gids = lax.broadcasted_iota(jnp.int32, (RB, G), 1)
    mask = (bt == gids).astype(jnp.float32)

    @pl.when(i == 0)
    def _():
        gs_ref[...] = jnp.zeros_like(gs_ref)
        c_ref[...] = jnp.zeros_like(c_ref)

    gs_ref[...] += lax.dot_general(mask, h, (((0,), (0,)), ((), ())),
                                   preferred_element_type=jnp.float32)
    c_ref[...] += lax.dot_general(mask, jnp.ones((RB, 8), jnp.float32),
                                  (((0,), (0,)), ((), ())),
                                  preferred_element_type=jnp.float32)


def _mm3(aggs, h2s, sdinv8, batch8, Wp, bp):
    return pl.pallas_call(
        _mm3_body,
        grid=(NBLK,),
        in_specs=[pl.BlockSpec((RB, F1), lambda i: (i, 0))] * 4
        + [pl.BlockSpec((RB, F1W), lambda i: (i, 0))] * 4 + [
            pl.BlockSpec((RB, 8), lambda i: (i, 0)),
            pl.BlockSpec((RB, 8), lambda i: (i, 0)),
            pl.BlockSpec((F2, F3), lambda i: (0, 0)),
            pl.BlockSpec((8, F3), lambda i: (0, 0)),
        ],
        out_specs=[
            pl.BlockSpec((RB, F3), lambda i: (i, 0)),
            pl.BlockSpec((G, F3), lambda i: (0, 0)),
            pl.BlockSpec((G, 8), lambda i: (0, 0)),
        ],
        out_shape=[
            jax.ShapeDtypeStruct((NPAD, F3), jnp.float32),
            jax.ShapeDtypeStruct((G, F3), jnp.float32),
            jax.ShapeDtypeStruct((G, 8), jnp.float32),
        ],
    )(*aggs, *h2s, sdinv8, batch8, Wp, bp)


def _head_body(gmx_ref, gs_ref, c_ref, f_ref, sb_ref, sa_ref, v_ref,
               Wg1, bg1, Wg2, bg2, Wf, bf, Wsb, bsb, Wsa, bsa, Wv, bv,
               W1, b1, W2, b2, Wo, bo, o_ref):
    gmax = jnp.max(gmx_ref[...], axis=0)
    cnt = c_ref[:, 0:1]
    gmean = gs_ref[...] / jnp.maximum(cnt, 1.0)
    g = jnp.concatenate([gmax[:, :780], gmean[:, :780]], axis=1)
    g = jnp.maximum(jnp.dot(g, Wg1[...], preferred_element_type=jnp.float32)
                    + bg1[0:1, :], 0.0)
    g = jnp.dot(g, Wg2[...], preferred_element_type=jnp.float32) + bg2[0:1, :]
    ff = jnp.dot(f_ref[...], Wf[...], preferred_element_type=jnp.float32) + bf[0:1, :]
    sb = jnp.dot(sb_ref[...], Wsb[...], preferred_element_type=jnp.float32) + bsb[0:1, :]
    sa = jnp.dot(sa_ref[...], Wsa[...], preferred_element_type=jnp.float32) + bsa[0:1, :]
    vv = jnp.dot(v_ref[...], Wv[...], preferred_element_type=jnp.float32) + bv[0:1, :]
    xc = jnp.concatenate([g, ff, sb, sa, vv], axis=1)
    xc = jnp.maximum(jnp.dot(xc, W1[...], preferred_element_type=jnp.float32)
                     + b1[0:1, :], 0.0)
    xc = jnp.maximum(jnp.dot(xc, W2[...], preferred_element_type=jnp.float32)
                     + b2[0:1, :], 0.0)
    o_ref[...] = jnp.dot(xc, Wo[...], preferred_element_type=jnp.float32) + bo[0:1, :]


def _head(gmax_parts, gsum, cnt8, fingerprint, seqbefore, seqafter, variant, wb):
    return pl.pallas_call(
        _head_body,
        out_shape=jax.ShapeDtypeStruct((G, 8), jnp.float32),
    )(gmax_parts, gsum, cnt8, fingerprint, seqbefore, seqafter, variant, *wb)


def _pad2(a, rows, cols):
    return jnp.pad(a, ((0, rows - a.shape[0]), (0, cols - a.shape[1])))


def _pad_bias(b, cols):
    return jnp.broadcast_to(jnp.pad(b, (0, cols - b.shape[0]))[None, :], (8, cols))


def kernel(x, edge_index, batch, fingerprint, seqbefore, seqafter, variant,
           W_c1, b_c1, W_c2, b_c2, W_c3, b_c3, W_g1, b_g1, W_g2, b_g2,
           W_f, b_f, W_sb, b_sb, W_sa, b_sa, W_v, b_v,
           W_1, b_1, W_2, b_2, W_o, b_o):
    xp = _pad2(x, NPAD, F1W)
    rows_real = jnp.arange(NPAD) < N
    xp = xp.at[:, 78].set(rows_real.astype(jnp.float32))
    batchp = jnp.pad(batch, (0, NPAD - N), constant_values=G)
    batch8 = jnp.broadcast_to(batchp[:, None], (NPAD, 8))
    node_idx = jnp.pad(edge_index[0], (0, EPAD - E), constant_values=-1)
    he_idx = jnp.pad(edge_index[1], (0, EPAD - E), constant_values=-1)

    s1 = _agg(xp, he_idx, node_idx)
    degA = s1[:, 78]
    binv, _u1, sdinv = _scales(degA, degA)
    binv8 = jnp.broadcast_to(binv[:, None], (NPAD, 8))
    e1 = _escale(s1, binv8)
    s2 = _agg(e1, node_idx, he_idx)
    degB = s2[:, 78]
    _u2, dinv, _u3 = _scales(degA, degB)
    dinv8 = jnp.broadcast_to(dinv[:, None], (NPAD, 8))
    sdinv8 = jnp.broadcast_to(sdinv[:, None], (NPAD, 8))

    h1 = _mm1(s2, dinv8, _pad2(W_c1, F1, F1W), _pad_bias(b_c1, F1W))
    e2 = _escale(_agg(h1, he_idx, node_idx), binv8)
    s4 = _agg(e2, node_idx, he_idx)
    h2s = _mm2(s4, dinv8, sdinv8, _pad2(W_c2, F1, F2), _pad_bias(b_c2, F2))
    aggs = [_agg(h2s[p], he_idx, node_idx) for p in range(4)]
    h3, gsum, cnt8 = _mm3(aggs, h2s, sdinv8, batch8,
                          _pad2(W_c3, F2, F3), _pad_bias(b_c3, F3))

    gmax_parts = _pool(h3, batchp).reshape(NSC * NTILE, G, F3)

    wb = [
        W_g1, _pad_bias(b_g1, 1500), W_g2, _pad_bias(b_g2, 128),
        W_f, _pad_bias(b_f, 128), W_sb, _pad_bias(b_sb, 128),
        W_sa, _pad_bias(b_sa, 128), W_v, _pad_bias(b_v, 384),
        W_1, _pad_bias(b_1, 512), W_2, _pad_bias(b_2, 128),
        _pad2(W_o, 128, 8), _pad_bias(b_o, 8),
    ]
    out = _head(gmax_parts, gsum, cnt8, fingerprint, seqbefore, seqafter,
                variant, wb)
    return out[:, :2]


# speedup vs baseline: 4.7020x; 1.5524x over previous
import jax
import jax.numpy as jnp
from jax import lax
from jax.experimental import pallas as pl
from jax.experimental.pallas import tpu as pltpu
from jax.experimental.pallas import tpu_sc as plsc

N = 50000
E = 800000
G = 128
NPAD = 50176
RB = 512
NBLK = NPAD // RB
F1 = 80
F1W = 80
F2 = 320
F3 = 784

NSC = 2
NTILE = 16
HALF = NPAD // 2
TROW = HALF // NTILE
EPAD = 801280
EPT = EPAD // NTILE
K = 32
IB = 160

_SC_MESH = dict(core_axis_name="c", subcore_axis_name="s")


def _agg_body(feat_hbm, scat_hbm, gath_hbm, out_hbm,
              sblk, gblk, sidx0, gidx0, sidx1, gidx1, rows0, rows1,
              acc, sem0, sem1):
    c = lax.axis_index("c")
    s = lax.axis_index("s")
    ebase = s * EPT
    nbase = c * HALF
    lane = lax.iota(jnp.int32, 16)

    for r in range(16):
        for k0 in range(F1 // 16):
            rows0[r, pl.ds(k0 * 16, 16)] = jnp.zeros((16,), jnp.float32)

    def _z(i, _):
        pltpu.sync_copy(rows0.at[pl.ds(0, 16)], acc.at[pl.ds(s * TROW + i * 16, 16)])
        return 0
    lax.fori_loop(0, TROW // 16, _z, 0)
    plsc.subcore_barrier()

    NCB = IB // K

    def _block(b, _):
        base = ebase + b * IB
        pltpu.sync_copy(scat_hbm.at[pl.ds(base, IB)], sblk)
        pltpu.sync_copy(gath_hbm.at[pl.ds(base, IB)], gblk)
        cps = [None] * NCB
        for t in range(NCB):
            sx, gx = (sidx0, gidx0) if t % 2 == 0 else (sidx1, gidx1)
            rw, sm = (rows0, sem0) if t % 2 == 0 else (rows1, sem1)
            for u in range(K // 16):
                sk = sblk[pl.ds(t * K + u * 16, 16)]
                gk = gblk[pl.ds(t * K + u * 16, 16)]
                loc = sk - nbase
                mi = lax.shift_right_arithmetic(
                    jnp.bitwise_or(loc, (HALF - 1) - loc), 31) + 1
                sx[pl.ds(u * 16, 16)] = jnp.minimum(
                    jnp.maximum(loc, 0), HALF - 1)
                zsp = N + (b * NCB + t * 2 + u) % 11 * 16
                gx[pl.ds(u * 16, 16)] = mi * gk + (1 - mi) * (zsp + lane)
            cps[t] = pltpu.async_copy(feat_hbm.at[gx], rw, sm)
            if t > 0:
                cps[t - 1].wait()
                pv = rows0 if (t - 1) % 2 == 0 else rows1
                px = sidx0 if (t - 1) % 2 == 0 else sidx1
                pltpu.sync_copy(pv, acc.at[px], add=True)
        cps[NCB - 1].wait()
        pv = rows0 if (NCB - 1) % 2 == 0 else rows1
        px = sidx0 if (NCB - 1) % 2 == 0 else sidx1
        pltpu.sync_copy(pv, acc.at[px], add=True)
        return 0
    lax.fori_loop(0, EPT // IB, _block, 0)
    plsc.subcore_barrier()

    g0 = c * HALF + s * TROW

    def _d(t, _):
        pltpu.sync_copy(acc.at[pl.ds(s * TROW + t * 16, 16)], rows0.at[pl.ds(0, 16)])
        pltpu.sync_copy(rows0.at[pl.ds(0, 16)], out_hbm.at[pl.ds(g0 + t * 16, 16)])
        return 0
    lax.fori_loop(0, TROW // 16, _d, 0)


def _agg(feat, scat, gath):
    f = pl.kernel(
        _agg_body,
        out_type=jax.ShapeDtypeStruct((NPAD, F1), jnp.float32),
        mesh=plsc.VectorSubcoreMesh(**_SC_MESH),
        compiler_params=pltpu.CompilerParams(use_tc_tiling_on_sc=False),
        scratch_types=[
            pltpu.VMEM((IB,), jnp.int32),
            pltpu.VMEM((IB,), jnp.int32),
            pltpu.VMEM((K,), jnp.int32),
            pltpu.VMEM((K,), jnp.int32),
            pltpu.VMEM((K,), jnp.int32),
            pltpu.VMEM((K,), jnp.int32),
            pltpu.VMEM((K, F1W), jnp.float32),
            pltpu.VMEM((K, F1W), jnp.float32),
            pltpu.VMEM_SHARED((HALF, F1), jnp.float32),
            pltpu.SemaphoreType.DMA,
            pltpu.SemaphoreType.DMA,
        ],
    )
    return f(feat, scat, gath)


def _pool_body(h3_hbm, batch_hbm, out_hbm, rowbuf, bbuf, acc):
    c = lax.axis_index("c")
    s = lax.axis_index("s")
    g0 = c * HALF + s * TROW

    def _z(i, _):
        acc[pl.ds(i * 16, 16)] = jnp.zeros((16,), jnp.float32)
        return 0
    lax.fori_loop(0, (G + 1) * F3 // 16, _z, 0)

    pltpu.sync_copy(batch_hbm.at[pl.ds(g0, TROW)], bbuf)

    def _grp(i, _):
        pltpu.sync_copy(h3_hbm.at[pl.ds(g0 + i * 16, 16)], rowbuf)
        bv = bbuf[pl.ds(i * 16, 16)]
        for r in range(16):
            base = bv[r] * F3

            def _col(k0, _):
                o = k0 * 16
                acc[pl.ds(base + o, 16)] = jnp.maximum(
                    acc[pl.ds(base + o, 16)], rowbuf[r, pl.ds(o, 16)])
                return 0
            lax.fori_loop(0, F3 // 16, _col, 0)
        return 0
    lax.fori_loop(0, TROW // 16, _grp, 0)

    pltpu.sync_copy(acc.at[pl.ds(0, G * F3)],
                    out_hbm.at[pl.ds((c * NTILE + s) * G * F3, G * F3)])


def _pool(h3, batchp):
    f = pl.kernel(
        _pool_body,
        out_type=jax.ShapeDtypeStruct((NSC * NTILE * G * F3,), jnp.float32),
        mesh=plsc.VectorSubcoreMesh(**_SC_MESH),
        compiler_params=pltpu.CompilerParams(use_tc_tiling_on_sc=False),
        scratch_types=[
            pltpu.VMEM((16, F3), jnp.float32),
            pltpu.VMEM((TROW,), jnp.int32),
            pltpu.VMEM(((G + 1) * F3,), jnp.float32),
        ],
    )
    return f(h3, batchp)


def _scales_body(degA_ref, degB_ref, binv_ref, dinv_ref, sdinv_ref):
    da = degA_ref[...]
    db = degB_ref[...]
    binv_ref[...] = jnp.where(da > 0, 1.0 / jnp.where(da > 0, da, 1.0), 0.0)
    dinv_ref[...] = jnp.where(db > 0, 1.0 / jnp.where(db > 0, db, 1.0), 0.0)
    sdinv_ref[...] = lax.rsqrt(da + 1.0)


def _scales(degA, degB):
    f = pl.pallas_call(
        _scales_body,
        out_shape=[jax.ShapeDtypeStruct((NPAD // 128, 128), jnp.float32)] * 3,
    )
    binv, dinv, sdinv = f(degA.reshape(NPAD // 128, 128),
                          degB.reshape(NPAD // 128, 128))
    return binv.reshape(NPAD), dinv.reshape(NPAD), sdinv.reshape(NPAD)


def _row_mask(i):
    rid = i * RB + lax.broadcasted_iota(jnp.int32, (RB, 1), 0)
    return (rid < N).astype(jnp.float32)


def _escale_body(a_ref, s_ref, o_ref):
    o_ref[...] = a_ref[...] * s_ref[:, 0:1]


def _escale(a, scale8):
    return pl.pallas_call(
        _escale_body,
        grid=(NBLK,),
        in_specs=[
            pl.BlockSpec((RB, F1), lambda i: (i, 0)),
            pl.BlockSpec((RB, 8), lambda i: (i, 0)),
        ],
        out_specs=pl.BlockSpec((RB, F1W), lambda i: (i, 0)),
        out_shape=jax.ShapeDtypeStruct((NPAD, F1W), jnp.float32),
    )(a, scale8)


def _mm1_body(a_ref, d_ref, W_ref, b_ref, o_ref):
    a = a_ref[...] * d_ref[:, 0:1]
    h = jnp.maximum(
        jnp.dot(a, W_ref[...], preferred_element_type=jnp.float32)
        + b_ref[0:1, :], 0.0)
    o_ref[...] = h * _row_mask(pl.program_id(0))


def _mm1(s2, dinv8, Wp, bp):
    return pl.pallas_call(
        _mm1_body,
        grid=(NBLK,),
        in_specs=[
            pl.BlockSpec((RB, F1), lambda i: (i, 0)),
            pl.BlockSpec((RB, 8), lambda i: (i, 0)),
            pl.BlockSpec((F1, F1W), lambda i: (0, 0)),
            pl.BlockSpec((8, F1W), lambda i: (0, 0)),
        ],
        out_specs=pl.BlockSpec((RB, F1W), lambda i: (i, 0)),
        out_shape=jax.ShapeDtypeStruct((NPAD, F1W), jnp.float32),
    )(s2, dinv8, Wp, bp)


def _mm2_body(a_ref, d_ref, s_ref, W_ref, b_ref, o0, o1, o2, o3):
    a = a_ref[...] * d_ref[:, 0:1]
    h = jnp.maximum(
        jnp.dot(a, W_ref[...], preferred_element_type=jnp.float32)
        + b_ref[0:1, :], 0.0)
    h = h * s_ref[:, 0:1] * _row_mask(pl.program_id(0))
    o0[...] = h[:, 0:F1]
    o1[...] = h[:, F1:2 * F1]
    o2[...] = h[:, 2 * F1:3 * F1]
    o3[...] = h[:, 3 * F1:4 * F1]


def _mm2(s4, dinv8, sdinv8, Wp, bp):
    return pl.pallas_call(
        _mm2_body,
        grid=(NBLK,),
        in_specs=[
            pl.BlockSpec((RB, F1), lambda i: (i, 0)),
            pl.BlockSpec((RB, 8), lambda i: (i, 0)),
            pl.BlockSpec((RB, 8), lambda i: (i, 0)),
            pl.BlockSpec((F1, F2), lambda i: (0, 0)),
            pl.BlockSpec((8, F2), lambda i: (0, 0)),
        ],
        out_specs=[pl.BlockSpec((RB, F1W), lambda i: (i, 0))] * 4,
        out_shape=[jax.ShapeDtypeStruct((NPAD, F1W), jnp.float32)] * 4,
    )(s4, dinv8, sdinv8, Wp, bp)


def _mm3_body(a0, a1, a2, a3r, h0, h1, h2, h3r, sd_ref, bt_ref, W_ref, b_ref,
              h3_ref, gs_ref, c_ref):
    i = pl.program_id(0)
    sd = sd_ref[:, 0:1]
    a = jnp.concatenate([(a0[...] + h0[...][:, :F1]) * sd,
                         (a1[...] + h1[...][:, :F1]) * sd,
                         (a2[...] + h2[...][:, :F1]) * sd,
                         (a3r[...] + h3r[...][:, :F1]) * sd], axis=1)
    h = jnp.maximum(
        jnp.dot(a, W_ref[...], preferred_element_type=jnp.float32)
        + b_ref[0:1, :], 0.0)
    h3_ref[...] = h
    bt = bt_ref[:, 0:1]
    gids = lax.broadcasted_iota(jnp.int32, (RB, G), 1)
    mask = (bt == gids).astype(jnp.float32)

    @pl.when(i == 0)
    def _():
        gs_ref[...] = jnp.zeros_like(gs_ref)
        c_ref[...] = jnp.zeros_like(c_ref)

    gs_ref[...] += lax.dot_general(mask, h, (((0,), (0,)), ((), ())),
                                   preferred_element_type=jnp.float32)
    c_ref[...] += lax.dot_general(mask, jnp.ones((RB, 8), jnp.float32),
                                  (((0,), (0,)), ((), ())),
                                  preferred_element_type=jnp.float32)


def _mm3(aggs, h2s, sdinv8, batch8, Wp, bp):
    return pl.pallas_call(
        _mm3_body,
        grid=(NBLK,),
        in_specs=[pl.BlockSpec((RB, F1), lambda i: (i, 0))] * 4
        + [pl.BlockSpec((RB, F1W), lambda i: (i, 0))] * 4 + [
            pl.BlockSpec((RB, 8), lambda i: (i, 0)),
            pl.BlockSpec((RB, 8), lambda i: (i, 0)),
            pl.BlockSpec((F2, F3), lambda i: (0, 0)),
            pl.BlockSpec((8, F3), lambda i: (0, 0)),
        ],
        out_specs=[
            pl.BlockSpec((RB, F3), lambda i: (i, 0)),
            pl.BlockSpec((G, F3), lambda i: (0, 0)),
            pl.BlockSpec((G, 8), lambda i: (0, 0)),
        ],
        out_shape=[
            jax.ShapeDtypeStruct((NPAD, F3), jnp.float32),
            jax.ShapeDtypeStruct((G, F3), jnp.float32),
            jax.ShapeDtypeStruct((G, 8), jnp.float32),
        ],
    )(*aggs, *h2s, sdinv8, batch8, Wp, bp)


def _head_body(gmx_ref, gs_ref, c_ref, f_ref, sb_ref, sa_ref, v_ref,
               Wg1, bg1, Wg2, bg2, Wf, bf, Wsb, bsb, Wsa, bsa, Wv, bv,
               W1, b1, W2, b2, Wo, bo, o_ref):
    gmax = jnp.max(gmx_ref[...], axis=0)
    cnt = c_ref[:, 0:1]
    gmean = gs_ref[...] / jnp.maximum(cnt, 1.0)
    g = jnp.concatenate([gmax[:, :780], gmean[:, :780]], axis=1)
    g = jnp.maximum(jnp.dot(g, Wg1[...], preferred_element_type=jnp.float32)
                    + bg1[0:1, :], 0.0)
    g = jnp.dot(g, Wg2[...], preferred_element_type=jnp.float32) + bg2[0:1, :]
    ff = jnp.dot(f_ref[...], Wf[...], preferred_element_type=jnp.float32) + bf[0:1, :]
    sb = jnp.dot(sb_ref[...], Wsb[...], preferred_element_type=jnp.float32) + bsb[0:1, :]
    sa = jnp.dot(sa_ref[...], Wsa[...], preferred_element_type=jnp.float32) + bsa[0:1, :]
    vv = jnp.dot(v_ref[...], Wv[...], preferred_element_type=jnp.float32) + bv[0:1, :]
    xc = jnp.concatenate([g, ff, sb, sa, vv], axis=1)
    xc = jnp.maximum(jnp.dot(xc, W1[...], preferred_element_type=jnp.float32)
                     + b1[0:1, :], 0.0)
    xc = jnp.maximum(jnp.dot(xc, W2[...], preferred_element_type=jnp.float32)
                     + b2[0:1, :], 0.0)
    o_ref[...] = jnp.dot(xc, Wo[...], preferred_element_type=jnp.float32) + bo[0:1, :]


def _head(gmax_parts, gsum, cnt8, fingerprint, seqbefore, seqafter, variant, wb):
    return pl.pallas_call(
        _head_body,
        out_shape=jax.ShapeDtypeStruct((G, 8), jnp.float32),
    )(gmax_parts, gsum, cnt8, fingerprint, seqbefore, seqafter, variant, *wb)


def _pad2(a, rows, cols):
    return jnp.pad(a, ((0, rows - a.shape[0]), (0, cols - a.shape[1])))


def _pad_bias(b, cols):
    return jnp.broadcast_to(jnp.pad(b, (0, cols - b.shape[0]))[None, :], (8, cols))


def kernel(x, edge_index, batch, fingerprint, seqbefore, seqafter, variant,
           W_c1, b_c1, W_c2, b_c2, W_c3, b_c3, W_g1, b_g1, W_g2, b_g2,
           W_f, b_f, W_sb, b_sb, W_sa, b_sa, W_v, b_v,
           W_1, b_1, W_2, b_2, W_o, b_o):
    xp = _pad2(x, NPAD, F1W)
    rows_real = jnp.arange(NPAD) < N
    xp = xp.at[:, 78].set(rows_real.astype(jnp.float32))
    batchp = jnp.pad(batch, (0, NPAD - N), constant_values=G)
    batch8 = jnp.broadcast_to(batchp[:, None], (NPAD, 8))
    node_idx = jnp.pad(edge_index[0], (0, EPAD - E), constant_values=-1)
    he_idx = jnp.pad(edge_index[1], (0, EPAD - E), constant_values=-1)

    s1 = _agg(xp, he_idx, node_idx)
    degA = s1[:, 78]
    binv, _u1, sdinv = _scales(degA, degA)
    binv8 = jnp.broadcast_to(binv[:, None], (NPAD, 8))
    e1 = _escale(s1, binv8)
    s2 = _agg(e1, node_idx, he_idx)
    degB = s2[:, 78]
    _u2, dinv, _u3 = _scales(degA, degB)
    dinv8 = jnp.broadcast_to(dinv[:, None], (NPAD, 8))
    sdinv8 = jnp.broadcast_to(sdinv[:, None], (NPAD, 8))

    h1 = _mm1(s2, dinv8, _pad2(W_c1, F1, F1W), _pad_bias(b_c1, F1W))
    e2 = _escale(_agg(h1, he_idx, node_idx), binv8)
    s4 = _agg(e2, node_idx, he_idx)
    h2s = _mm2(s4, dinv8, sdinv8, _pad2(W_c2, F1, F2), _pad_bias(b_c2, F2))
    aggs = [_agg(h2s[p], he_idx, node_idx) for p in range(4)]
    h3, gsum, cnt8 = _mm3(aggs, h2s, sdinv8, batch8,
                          _pad2(W_c3, F2, F3), _pad_bias(b_c3, F3))

    gmax_parts = _pool(h3, batchp).reshape(NSC * NTILE, G, F3)

    wb = [
        W_g1, _pad_bias(b_g1, 1500), W_g2, _pad_bias(b_g2, 128),
        W_f, _pad_bias(b_f, 128), W_sb, _pad_bias(b_sb, 128),
        W_sa, _pad_bias(b_sa, 128), W_v, _pad_bias(b_v, 384),
        W_1, _pad_bias(b_1, 512), W_2, _pad_bias(b_2, 128),
        _pad2(W_o, 128, 8), _pad_bias(b_o, 8),
    ]
    out = _head(gmax_parts, gsum, cnt8, fingerprint, seqbefore, seqafter,
                variant, wb)
    return out[:, :2]


# speedup vs baseline: 4.7158x; 1.0029x over previous
import jax
import jax.numpy as jnp
from jax import lax
from jax.experimental import pallas as pl
from jax.experimental.pallas import tpu as pltpu
from jax.experimental.pallas import tpu_sc as plsc

N = 50000
E = 800000
G = 128
NPAD = 50176
RB = 512
NBLK = NPAD // RB
F1 = 80
F1W = 80
F2 = 320
F3 = 784

NSC = 2
NTILE = 16
HALF = NPAD // 2
TROW = HALF // NTILE
EPAD = 801280
EPT = EPAD // NTILE
K = 32
IB = 160

_SC_MESH = dict(core_axis_name="c", subcore_axis_name="s")


def _agg_body(feat_hbm, scat_hbm, gath_hbm, out_hbm,
              sblk, gblk, sidx0, gidx0, sidx1, gidx1, rows0, rows1,
              acc, sem0, sem1, ssem0, ssem1):
    c = lax.axis_index("c")
    s = lax.axis_index("s")
    ebase = s * EPT
    nbase = c * HALF
    lane = lax.iota(jnp.int32, 16)

    for r in range(16):
        for k0 in range(F1 // 16):
            rows0[r, pl.ds(k0 * 16, 16)] = jnp.zeros((16,), jnp.float32)

    def _z(i, _):
        pltpu.sync_copy(rows0.at[pl.ds(0, 16)], acc.at[pl.ds(s * TROW + i * 16, 16)])
        return 0
    lax.fori_loop(0, TROW // 16, _z, 0)
    plsc.subcore_barrier()

    NCB = IB // K

    def _block(b, _):
        base = ebase + b * IB
        pltpu.sync_copy(scat_hbm.at[pl.ds(base, IB)], sblk)
        pltpu.sync_copy(gath_hbm.at[pl.ds(base, IB)], gblk)
        cps = [None] * NCB
        sps = [None] * NCB
        for t in range(NCB):
            sx, gx = (sidx0, gidx0) if t % 2 == 0 else (sidx1, gidx1)
            rw, sm = (rows0, sem0) if t % 2 == 0 else (rows1, sem1)
            ss = ssem0 if t % 2 == 1 else ssem1
            for u in range(K // 16):
                sk = sblk[pl.ds(t * K + u * 16, 16)]
                gk = gblk[pl.ds(t * K + u * 16, 16)]
                loc = sk - nbase
                mi = lax.shift_right_arithmetic(
                    jnp.bitwise_or(loc, (HALF - 1) - loc), 31) + 1
                sx[pl.ds(u * 16, 16)] = jnp.minimum(
                    jnp.maximum(loc, 0), HALF - 1)
                zsp = N + (b * NCB + t * 2 + u) % 11 * 16
                gx[pl.ds(u * 16, 16)] = mi * gk + (1 - mi) * (zsp + lane)
            if t > 1:
                sps[t - 2].wait()
            cps[t] = pltpu.async_copy(feat_hbm.at[gx], rw, sm)
            if t > 0:
                cps[t - 1].wait()
                pv = rows0 if (t - 1) % 2 == 0 else rows1
                px = sidx0 if (t - 1) % 2 == 0 else sidx1
                sps[t - 1] = pltpu.async_copy(pv, acc.at[px], ss, add=True)
        cps[NCB - 1].wait()
        pv = rows0 if (NCB - 1) % 2 == 0 else rows1
        px = sidx0 if (NCB - 1) % 2 == 0 else sidx1
        sps[NCB - 1] = pltpu.async_copy(pv, acc.at[px],
                                        ssem0 if (NCB - 1) % 2 == 0 else ssem1,
                                        add=True)
        sps[NCB - 2].wait()
        sps[NCB - 1].wait()
        return 0
    lax.fori_loop(0, EPT // IB, _block, 0)
    plsc.subcore_barrier()

    g0 = c * HALF + s * TROW

    def _d(t, _):
        pltpu.sync_copy(acc.at[pl.ds(s * TROW + t * 16, 16)], rows0.at[pl.ds(0, 16)])
        pltpu.sync_copy(rows0.at[pl.ds(0, 16)], out_hbm.at[pl.ds(g0 + t * 16, 16)])
        return 0
    lax.fori_loop(0, TROW // 16, _d, 0)


def _agg(feat, scat, gath):
    f = pl.kernel(
        _agg_body,
        out_type=jax.ShapeDtypeStruct((NPAD, F1), jnp.float32),
        mesh=plsc.VectorSubcoreMesh(**_SC_MESH),
        compiler_params=pltpu.CompilerParams(use_tc_tiling_on_sc=False),
        scratch_types=[
            pltpu.VMEM((IB,), jnp.int32),
            pltpu.VMEM((IB,), jnp.int32),
            pltpu.VMEM((K,), jnp.int32),
            pltpu.VMEM((K,), jnp.int32),
            pltpu.VMEM((K,), jnp.int32),
            pltpu.VMEM((K,), jnp.int32),
            pltpu.VMEM((K, F1W), jnp.float32),
            pltpu.VMEM((K, F1W), jnp.float32),
            pltpu.VMEM_SHARED((HALF, F1), jnp.float32),
            pltpu.SemaphoreType.DMA,
            pltpu.SemaphoreType.DMA,
            pltpu.SemaphoreType.DMA,
            pltpu.SemaphoreType.DMA,
        ],
    )
    return f(feat, scat, gath)


def _pool_body(h3_hbm, batch_hbm, out_hbm, rowbuf, bbuf, acc):
    c = lax.axis_index("c")
    s = lax.axis_index("s")
    g0 = c * HALF + s * TROW

    def _z(i, _):
        acc[pl.ds(i * 16, 16)] = jnp.zeros((16,), jnp.float32)
        return 0
    lax.fori_loop(0, (G + 1) * F3 // 16, _z, 0)

    pltpu.sync_copy(batch_hbm.at[pl.ds(g0, TROW)], bbuf)

    def _grp(i, _):
        pltpu.sync_copy(h3_hbm.at[pl.ds(g0 + i * 16, 16)], rowbuf)
        bv = bbuf[pl.ds(i * 16, 16)]
        for r in range(16):
            base = bv[r] * F3

            def _col(k0, _):
                o = k0 * 16
                acc[pl.ds(base + o, 16)] = jnp.maximum(
                    acc[pl.ds(base + o, 16)], rowbuf[r, pl.ds(o, 16)])
                return 0
            lax.fori_loop(0, F3 // 16, _col, 0)
        return 0
    lax.fori_loop(0, TROW // 16, _grp, 0)

    pltpu.sync_copy(acc.at[pl.ds(0, G * F3)],
                    out_hbm.at[pl.ds((c * NTILE + s) * G * F3, G * F3)])


def _pool(h3, batchp):
    f = pl.kernel(
        _pool_body,
        out_type=jax.ShapeDtypeStruct((NSC * NTILE * G * F3,), jnp.float32),
        mesh=plsc.VectorSubcoreMesh(**_SC_MESH),
        compiler_params=pltpu.CompilerParams(use_tc_tiling_on_sc=False),
        scratch_types=[
            pltpu.VMEM((16, F3), jnp.float32),
            pltpu.VMEM((TROW,), jnp.int32),
            pltpu.VMEM(((G + 1) * F3,), jnp.float32),
        ],
    )
    return f(h3, batchp)


def _scales_body(degA_ref, degB_ref, binv_ref, dinv_ref, sdinv_ref):
    da = degA_ref[...]
    db = degB_ref[...]
    binv_ref[...] = jnp.where(da > 0, 1.0 / jnp.where(da > 0, da, 1.0), 0.0)
    dinv_ref[...] = jnp.where(db > 0, 1.0 / jnp.where(db > 0, db, 1.0), 0.0)
    sdinv_ref[...] = lax.rsqrt(da + 1.0)


def _scales(degA, degB):
    f = pl.pallas_call(
        _scales_body,
        out_shape=[jax.ShapeDtypeStruct((NPAD // 128, 128), jnp.float32)] * 3,
    )
    binv, dinv, sdinv = f(degA.reshape(NPAD // 128, 128),
                          degB.reshape(NPAD // 128, 128))
    return binv.reshape(NPAD), dinv.reshape(NPAD), sdinv.reshape(NPAD)


def _row_mask(i):
    rid = i * RB + lax.broadcasted_iota(jnp.int32, (RB, 1), 0)
    return (rid < N).astype(jnp.float32)


def _escale_body(a_ref, s_ref, o_ref):
    o_ref[...] = a_ref[...] * s_ref[:, 0:1]


def _escale(a, scale8):
    return pl.pallas_call(
        _escale_body,
        grid=(NBLK,),
        in_specs=[
            pl.BlockSpec((RB, F1), lambda i: (i, 0)),
            pl.BlockSpec((RB, 8), lambda i: (i, 0)),
        ],
        out_specs=pl.BlockSpec((RB, F1W), lambda i: (i, 0)),
        out_shape=jax.ShapeDtypeStruct((NPAD, F1W), jnp.float32),
    )(a, scale8)


def _mm1_body(a_ref, d_ref, W_ref, b_ref, o_ref):
    a = a_ref[...] * d_ref[:, 0:1]
    h = jnp.maximum(
        jnp.dot(a, W_ref[...], preferred_element_type=jnp.float32)
        + b_ref[0:1, :], 0.0)
    o_ref[...] = h * _row_mask(pl.program_id(0))


def _mm1(s2, dinv8, Wp, bp):
    return pl.pallas_call(
        _mm1_body,
        grid=(NBLK,),
        in_specs=[
            pl.BlockSpec((RB, F1), lambda i: (i, 0)),
            pl.BlockSpec((RB, 8), lambda i: (i, 0)),
            pl.BlockSpec((F1, F1W), lambda i: (0, 0)),
            pl.BlockSpec((8, F1W), lambda i: (0, 0)),
        ],
        out_specs=pl.BlockSpec((RB, F1W), lambda i: (i, 0)),
        out_shape=jax.ShapeDtypeStruct((NPAD, F1W), jnp.float32),
    )(s2, dinv8, Wp, bp)


def _mm2_body(a_ref, d_ref, s_ref, W_ref, b_ref, o0, o1, o2, o3):
    a = a_ref[...] * d_ref[:, 0:1]
    h = jnp.maximum(
        jnp.dot(a, W_ref[...], preferred_element_type=jnp.float32)
        + b_ref[0:1, :], 0.0)
    h = h * s_ref[:, 0:1] * _row_mask(pl.program_id(0))
    o0[...] = h[:, 0:F1]
    o1[...] = h[:, F1:2 * F1]
    o2[...] = h[:, 2 * F1:3 * F1]
    o3[...] = h[:, 3 * F1:4 * F1]


def _mm2(s4, dinv8, sdinv8, Wp, bp):
    return pl.pallas_call(
        _mm2_body,
        grid=(NBLK,),
        in_specs=[
            pl.BlockSpec((RB, F1), lambda i: (i, 0)),
            pl.BlockSpec((RB, 8), lambda i: (i, 0)),
            pl.BlockSpec((RB, 8), lambda i: (i, 0)),
            pl.BlockSpec((F1, F2), lambda i: (0, 0)),
            pl.BlockSpec((8, F2), lambda i: (0, 0)),
        ],
        out_specs=[pl.BlockSpec((RB, F1W), lambda i: (i, 0))] * 4,
        out_shape=[jax.ShapeDtypeStruct((NPAD, F1W), jnp.float32)] * 4,
    )(s4, dinv8, sdinv8, Wp, bp)


def _mm3_body(a0, a1, a2, a3r, h0, h1, h2, h3r, sd_ref, bt_ref, W_ref, b_ref,
              h3_ref, gs_ref, c_ref):
    i = pl.program_id(0)
    sd = sd_ref[:, 0:1]
    a = jnp.concatenate([(a0[...] + h0[...][:, :F1]) * sd,
                         (a1[...] + h1[...][:, :F1]) * sd,
                         (a2[...] + h2[...][:, :F1]) * sd,
                         (a3r[...] + h3r[...][:, :F1]) * sd], axis=1)
    h = jnp.maximum(
        jnp.dot(a, W_ref[...], preferred_element_type=jnp.float32)
        + b_ref[0:1, :], 0.0)
    h3_ref[...] = h
    bt = bt_ref[:, 0:1]
    gids = lax.broadcasted_iota(jnp.int32, (RB, G), 1)
    mask = (bt == gids).astype(jnp.float32)

    @pl.when(i == 0)
    def _():
        gs_ref[...] = jnp.zeros_like(gs_ref)
        c_ref[...] = jnp.zeros_like(c_ref)

    gs_ref[...] += lax.dot_general(mask, h, (((0,), (0,)), ((), ())),
                                   preferred_element_type=jnp.float32)
    c_ref[...] += lax.dot_general(mask, jnp.ones((RB, 8), jnp.float32),
                                  (((0,), (0,)), ((), ())),
                                  preferred_element_type=jnp.float32)


def _mm3(aggs, h2s, sdinv8, batch8, Wp, bp):
    return pl.pallas_call(
        _mm3_body,
        grid=(NBLK,),
        in_specs=[pl.BlockSpec((RB, F1), lambda i: (i, 0))] * 4
        + [pl.BlockSpec((RB, F1W), lambda i: (i, 0))] * 4 + [
            pl.BlockSpec((RB, 8), lambda i: (i, 0)),
            pl.BlockSpec((RB, 8), lambda i: (i, 0)),
            pl.BlockSpec((F2, F3), lambda i: (0, 0)),
            pl.BlockSpec((8, F3), lambda i: (0, 0)),
        ],
        out_specs=[
            pl.BlockSpec((RB, F3), lambda i: (i, 0)),
            pl.BlockSpec((G, F3), lambda i: (0, 0)),
            pl.BlockSpec((G, 8), lambda i: (0, 0)),
        ],
        out_shape=[
            jax.ShapeDtypeStruct((NPAD, F3), jnp.float32),
            jax.ShapeDtypeStruct((G, F3), jnp.float32),
            jax.ShapeDtypeStruct((G, 8), jnp.float32),
        ],
    )(*aggs, *h2s, sdinv8, batch8, Wp, bp)


def _head_body(gmx_ref, gs_ref, c_ref, f_ref, sb_ref, sa_ref, v_ref,
               Wg1, bg1, Wg2, bg2, Wf, bf, Wsb, bsb, Wsa, bsa, Wv, bv,
               W1, b1, W2, b2, Wo, bo, o_ref):
    gmax = jnp.max(gmx_ref[...], axis=0)
    cnt = c_ref[:, 0:1]
    gmean = gs_ref[...] / jnp.maximum(cnt, 1.0)
    g = jnp.concatenate([gmax[:, :780], gmean[:, :780]], axis=1)
    g = jnp.maximum(jnp.dot(g, Wg1[...], preferred_element_type=jnp.float32)
                    + bg1[0:1, :], 0.0)
    g = jnp.dot(g, Wg2[...], preferred_element_type=jnp.float32) + bg2[0:1, :]
    ff = jnp.dot(f_ref[...], Wf[...], preferred_element_type=jnp.float32) + bf[0:1, :]
    sb = jnp.dot(sb_ref[...], Wsb[...], preferred_element_type=jnp.float32) + bsb[0:1, :]
    sa = jnp.dot(sa_ref[...], Wsa[...], preferred_element_type=jnp.float32) + bsa[0:1, :]
    vv = jnp.dot(v_ref[...], Wv[...], preferred_element_type=jnp.float32) + bv[0:1, :]
    xc = jnp.concatenate([g, ff, sb, sa, vv], axis=1)
    xc = jnp.maximum(jnp.dot(xc, W1[...], preferred_element_type=jnp.float32)
                     + b1[0:1, :], 0.0)
    xc = jnp.maximum(jnp.dot(xc, W2[...], preferred_element_type=jnp.float32)
                     + b2[0:1, :], 0.0)
    o_ref[...] = jnp.dot(xc, Wo[...], preferred_element_type=jnp.float32) + bo[0:1, :]


def _head(gmax_parts, gsum, cnt8, fingerprint, seqbefore, seqafter, variant, wb):
    return pl.pallas_call(
        _head_body,
        out_shape=jax.ShapeDtypeStruct((G, 8), jnp.float32),
    )(gmax_parts, gsum, cnt8, fingerprint, seqbefore, seqafter, variant, *wb)


def _pad2(a, rows, cols):
    return jnp.pad(a, ((0, rows - a.shape[0]), (0, cols - a.shape[1])))


def _pad_bias(b, cols):
    return jnp.broadcast_to(jnp.pad(b, (0, cols - b.shape[0]))[None, :], (8, cols))


def kernel(x, edge_index, batch, fingerprint, seqbefore, seqafter, variant,
           W_c1, b_c1, W_c2, b_c2, W_c3, b_c3, W_g1, b_g1, W_g2, b_g2,
           W_f, b_f, W_sb, b_sb, W_sa, b_sa, W_v, b_v,
           W_1, b_1, W_2, b_2, W_o, b_o):
    xp = _pad2(x, NPAD, F1W)
    rows_real = jnp.arange(NPAD) < N
    xp = xp.at[:, 78].set(rows_real.astype(jnp.float32))
    batchp = jnp.pad(batch, (0, NPAD - N), constant_values=G)
    batch8 = jnp.broadcast_to(batchp[:, None], (NPAD, 8))
    node_idx = jnp.pad(edge_index[0], (0, EPAD - E), constant_values=-1)
    he_idx = jnp.pad(edge_index[1], (0, EPAD - E), constant_values=-1)

    s1 = _agg(xp, he_idx, node_idx)
    degA = s1[:, 78]
    binv, _u1, sdinv = _scales(degA, degA)
    binv8 = jnp.broadcast_to(binv[:, None], (NPAD, 8))
    e1 = _escale(s1, binv8)
    s2 = _agg(e1, node_idx, he_idx)
    degB = s2[:, 78]
    _u2, dinv, _u3 = _scales(degA, degB)
    dinv8 = jnp.broadcast_to(dinv[:, None], (NPAD, 8))
    sdinv8 = jnp.broadcast_to(sdinv[:, None], (NPAD, 8))

    h1 = _mm1(s2, dinv8, _pad2(W_c1, F1, F1W), _pad_bias(b_c1, F1W))
    e2 = _escale(_agg(h1, he_idx, node_idx), binv8)
    s4 = _agg(e2, node_idx, he_idx)
    h2s = _mm2(s4, dinv8, sdinv8, _pad2(W_c2, F1, F2), _pad_bias(b_c2, F2))
    aggs = [_agg(h2s[p], he_idx, node_idx) for p in range(4)]
    h3, gsum, cnt8 = _mm3(aggs, h2s, sdinv8, batch8,
                          _pad2(W_c3, F2, F3), _pad_bias(b_c3, F3))

    gmax_parts = _pool(h3, batchp).reshape(NSC * NTILE, G, F3)

    wb = [
        W_g1, _pad_bias(b_g1, 1500), W_g2, _pad_bias(b_g2, 128),
        W_f, _pad_bias(b_f, 128), W_sb, _pad_bias(b_sb, 128),
        W_sa, _pad_bias(b_sa, 128), W_v, _pad_bias(b_v, 384),
        W_1, _pad_bias(b_1, 512), W_2, _pad_bias(b_2, 128),
        _pad2(W_o, 128, 8), _pad_bias(b_o, 8),
    ]
    out = _head(gmax_parts, gsum, cnt8, fingerprint, seqbefore, seqafter,
                variant, wb)
    return out[:, :2]
